# speedup vs baseline: 7.5716x; 7.5716x over previous
import jax
import jax.numpy as jnp
from jax.experimental import pallas as pl
from jax.experimental.pallas import tpu as pltpu
from jax.experimental.pallas import tpu_sc as plsc


def _project_table(table, W, b2):
    V, D = table.shape
    F = W.shape[1]
    R = 4000

    def body(t_ref, w_ref, b_ref, o_ref):
        acc = jnp.dot(t_ref[...], w_ref[...], preferred_element_type=jnp.float32)
        o_ref[...] = jnp.tanh(acc + b_ref[...])

    return pl.pallas_call(
        body,
        grid=(V // R,),
        in_specs=[
            pl.BlockSpec((R, D), lambda i: (i, 0)),
            pl.BlockSpec((D, F), lambda i: (0, 0)),
            pl.BlockSpec((1, F), lambda i: (0, 0)),
        ],
        out_specs=pl.BlockSpec((R, F), lambda i: (i, 0)),
        out_shape=jax.ShapeDtypeStruct((V, F), jnp.float32),
    )(table, W, b2)


_GATHER_WINDOW = 128


def _sc_gather(q, indices):
    n = indices.shape[1]
    F = q.shape[1]
    mesh = plsc.VectorSubcoreMesh(core_axis_name="core", subcore_axis_name="subcore")

    @pl.kernel(out_type=jax.ShapeDtypeStruct((n, F), q.dtype), mesh=mesh)
    def k(q_hbm, i_hbm, o_hbm):
        def body(i_vmem, o_vmem):
            pltpu.sync_copy(q_hbm.at[i_vmem.at[0]], o_vmem)

        pltpu.emit_pipeline(
            body,
            grid=(n // _GATHER_WINDOW,),
            in_specs=[pl.BlockSpec((1, _GATHER_WINDOW), lambda i: (0, i))],
            out_specs=[pl.BlockSpec((_GATHER_WINDOW, F), lambda i: (i, 0))],
            core_axis_name=("core", "subcore"),
            dimension_semantics=(pltpu.PARALLEL,),
        )(i_hbm, o_hbm)

    return k(q, indices)


def kernel(inputs, table, W, b):
    Bsz, L = inputs.shape
    F = W.shape[1]
    q = _project_table(table, W, b.reshape(1, F))
    flat = inputs.reshape(1, Bsz * L).astype(jnp.int32)
    out = _sc_gather(q, flat)
    return out.reshape(Bsz, L, F)


# speedup vs baseline: 9.1724x; 1.2114x over previous
import jax
import jax.numpy as jnp
from jax.experimental import pallas as pl
from jax.experimental.pallas import tpu as pltpu
from jax.experimental.pallas import tpu_sc as plsc


def _project_table(table, W, b2):
    V, D = table.shape
    F = W.shape[1]
    R = 4000

    def body(t_ref, w_ref, b_ref, o_ref):
        acc = jnp.dot(t_ref[...], w_ref[...], preferred_element_type=jnp.float32)
        o_ref[...] = jnp.tanh(acc + b_ref[...])

    return pl.pallas_call(
        body,
        grid=(V // R,),
        in_specs=[
            pl.BlockSpec((R, D), lambda i: (i, 0)),
            pl.BlockSpec((D, F), lambda i: (0, 0)),
            pl.BlockSpec((1, F), lambda i: (0, 0)),
        ],
        out_specs=pl.BlockSpec((R, F), lambda i: (i, 0)),
        out_shape=jax.ShapeDtypeStruct((V, F), jnp.float32),
    )(table, W, b2)


_GATHER_WINDOW = 256


def _sc_gather(q, indices):
    n = indices.shape[1]
    F = q.shape[1]
    mesh = plsc.VectorSubcoreMesh(core_axis_name="core", subcore_axis_name="subcore")

    @pl.kernel(out_type=jax.ShapeDtypeStruct((n, F), q.dtype), mesh=mesh)
    def k(q_hbm, i_hbm, o_hbm):
        def body(i_vmem, o_vmem):
            pltpu.sync_copy(q_hbm.at[i_vmem.at[0]], o_vmem)

        pltpu.emit_pipeline(
            body,
            grid=(n // _GATHER_WINDOW,),
            in_specs=[pl.BlockSpec((1, _GATHER_WINDOW), lambda i: (0, i))],
            out_specs=[pl.BlockSpec((_GATHER_WINDOW, F), lambda i: (i, 0))],
            core_axis_name=("core", "subcore"),
            dimension_semantics=(pltpu.PARALLEL,),
        )(i_hbm, o_hbm)

    return k(q, indices)


def kernel(inputs, table, W, b):
    Bsz, L = inputs.shape
    F = W.shape[1]
    q = _project_table(table, W, b.reshape(1, F))
    flat = inputs.reshape(1, Bsz * L).astype(jnp.int32)
    out = _sc_gather(q, flat)
    return out.reshape(Bsz, L, F)


# speedup vs baseline: 9.2790x; 1.0116x over previous
import functools

import jax
import jax.numpy as jnp
from jax import lax
from jax.experimental import pallas as pl
from jax.experimental.pallas import tpu as pltpu
from jax.experimental.pallas import tpu_sc as plsc


def _project_table(table, W, b2):
    V, D = table.shape
    F = W.shape[1]
    R = 4000

    def body(t_ref, w_ref, b_ref, o_ref):
        acc = jnp.dot(t_ref[...], w_ref[...], preferred_element_type=jnp.float32)
        o_ref[...] = jnp.tanh(acc + b_ref[...])

    return pl.pallas_call(
        body,
        grid=(V // R,),
        in_specs=[
            pl.BlockSpec((R, D), lambda i: (i, 0)),
            pl.BlockSpec((D, F), lambda i: (0, 0)),
            pl.BlockSpec((1, F), lambda i: (0, 0)),
        ],
        out_specs=pl.BlockSpec((R, F), lambda i: (i, 0)),
        out_shape=jax.ShapeDtypeStruct((V, F), jnp.float32),
    )(table, W, b2)


_NC = 2
_NS = 16
_NW = _NC * _NS
_CH = 128
_NBUF = 4


def _sc_gather(q, idx_flat):
    n = idx_flat.shape[0]
    F = q.shape[1]
    b_per_w = n // _NW
    n_ch = b_per_w // _CH
    mesh = plsc.VectorSubcoreMesh(core_axis_name="c", subcore_axis_name="s")

    @functools.partial(
        pl.kernel,
        mesh=mesh,
        out_type=jax.ShapeDtypeStruct((n, F), q.dtype),
        scratch_types=(
            [pltpu.VMEM((b_per_w,), jnp.int32)]
            + [pltpu.VMEM((_CH, F), jnp.float32) for _ in range(_NBUF)]
            + [pltpu.SemaphoreType.DMA for _ in range(2 * _NBUF)]
        ),
    )
    def k(q_hbm, i_hbm, o_hbm, idx_v, b0, b1, b2, b3, g0, g1, g2, g3, s0, s1, s2, s3):
        bufs = (b0, b1, b2, b3)
        gsem = (g0, g1, g2, g3)
        osem = (s0, s1, s2, s3)
        wid = lax.axis_index("s") * _NC + lax.axis_index("c")
        base = wid * b_per_w
        pltpu.sync_copy(i_hbm.at[pl.ds(base, b_per_w)], idx_v)

        def gather_start(c, j):
            pltpu.async_copy(q_hbm.at[idx_v.at[pl.ds(c * _CH, _CH)]], bufs[j], gsem[j])

        def gather_wait(j):
            pltpu.make_async_copy(
                q_hbm.at[idx_v.at[pl.ds(0, _CH)]], bufs[j], gsem[j]
            ).wait()

        def out_start(c, j):
            pltpu.async_copy(bufs[j], o_hbm.at[pl.ds(base + c * _CH, _CH)], osem[j])

        def out_wait(j):
            pltpu.make_async_copy(
                bufs[j], o_hbm.at[pl.ds(base, _CH)], osem[j]
            ).wait()

        gather_start(0, 0)
        gather_start(1, 1)
        gather_start(2, 2)
        gather_wait(0)
        out_start(0, 0)
        gather_start(3, 3)
        gather_wait(1)
        out_start(1, 1)
        out_wait(0)
        gather_start(4, 0)
        gather_wait(2)
        out_start(2, 2)
        out_wait(1)
        gather_start(5, 1)
        gather_wait(3)
        out_start(3, 3)

        @pl.loop(4, n_ch - 4, step=_NBUF)
        def _(c0):
            for j2 in range(_NBUF):
                i = c0 + j2
                j = (4 + j2) % _NBUF
                jn = (4 + j2 + 2) % _NBUF
                out_wait(jn)
                gather_start(i + 2, jn)
                gather_wait(j)
                out_start(i, j)

        out_wait(2)
        gather_start(n_ch - 2, 2)
        gather_wait(0)
        out_start(n_ch - 4, 0)
        out_wait(3)
        gather_start(n_ch - 1, 3)
        gather_wait(1)
        out_start(n_ch - 3, 1)
        gather_wait(2)
        out_start(n_ch - 2, 2)
        gather_wait(3)
        out_start(n_ch - 1, 3)
        out_wait(0)
        out_wait(1)
        out_wait(2)
        out_wait(3)

    return k(q, idx_flat)


def kernel(inputs, table, W, b):
    Bsz, L = inputs.shape
    F = W.shape[1]
    q = _project_table(table, W, b.reshape(1, F))
    flat = inputs.reshape(Bsz * L).astype(jnp.int32)
    out = _sc_gather(q, flat)
    return out.reshape(Bsz, L, F)


# speedup vs baseline: 9.2849x; 1.0006x over previous
import functools

import jax
import jax.numpy as jnp
from jax import lax
from jax.experimental import pallas as pl
from jax.experimental.pallas import tpu as pltpu
from jax.experimental.pallas import tpu_sc as plsc


def _project_table(table, W, b2):
    V, D = table.shape
    F = W.shape[1]
    R = 4000

    def body(t_ref, w_ref, b_ref, o_ref):
        acc = jnp.dot(t_ref[...], w_ref[...], preferred_element_type=jnp.float32)
        o_ref[...] = jnp.tanh(acc + b_ref[...])

    return pl.pallas_call(
        body,
        grid=(V // R,),
        in_specs=[
            pl.BlockSpec((R, D), lambda i: (i, 0)),
            pl.BlockSpec((D, F), lambda i: (0, 0)),
            pl.BlockSpec((1, F), lambda i: (0, 0)),
        ],
        out_specs=pl.BlockSpec((R, F), lambda i: (i, 0)),
        out_shape=jax.ShapeDtypeStruct((V, F), jnp.float32),
    )(table, W, b2)


_NC = 2
_NS = 16
_NW = _NC * _NS
_CH = 200
_NBUF = 4


def _sc_gather(q, idx_flat):
    n = idx_flat.shape[0]
    F = q.shape[1]
    b_per_w = n // _NW
    n_ch = b_per_w // _CH
    mesh = plsc.VectorSubcoreMesh(core_axis_name="c", subcore_axis_name="s")

    @functools.partial(
        pl.kernel,
        mesh=mesh,
        out_type=jax.ShapeDtypeStruct((n, F), q.dtype),
        scratch_types=(
            [pltpu.VMEM((b_per_w,), jnp.int32)]
            + [pltpu.VMEM((_CH, F), jnp.float32) for _ in range(_NBUF)]
            + [pltpu.SemaphoreType.DMA for _ in range(2 * _NBUF)]
        ),
    )
    def k(q_hbm, i_hbm, o_hbm, idx_v, b0, b1, b2, b3, g0, g1, g2, g3, s0, s1, s2, s3):
        bufs = (b0, b1, b2, b3)
        gsem = (g0, g1, g2, g3)
        osem = (s0, s1, s2, s3)
        wid = lax.axis_index("s") * _NC + lax.axis_index("c")
        base = wid * b_per_w
        pltpu.sync_copy(i_hbm.at[pl.ds(base, b_per_w)], idx_v)

        def gather_start(c, j):
            pltpu.async_copy(q_hbm.at[idx_v.at[pl.ds(c * _CH, _CH)]], bufs[j], gsem[j])

        def gather_wait(j):
            pltpu.make_async_copy(
                q_hbm.at[idx_v.at[pl.ds(0, _CH)]], bufs[j], gsem[j]
            ).wait()

        def out_start(c, j):
            pltpu.async_copy(bufs[j], o_hbm.at[pl.ds(base + c * _CH, _CH)], osem[j])

        def out_wait(j):
            pltpu.make_async_copy(
                bufs[j], o_hbm.at[pl.ds(base, _CH)], osem[j]
            ).wait()

        gather_start(0, 0)
        gather_start(1, 1)
        gather_start(2, 2)
        gather_wait(0)
        out_start(0, 0)
        gather_start(3, 3)
        gather_wait(1)
        out_start(1, 1)
        out_wait(0)
        gather_start(4, 0)
        gather_wait(2)
        out_start(2, 2)
        out_wait(1)
        gather_start(5, 1)
        gather_wait(3)
        out_start(3, 3)

        @pl.loop(4, n_ch - 4, step=_NBUF)
        def _(c0):
            for j2 in range(_NBUF):
                i = c0 + j2
                j = (4 + j2) % _NBUF
                jn = (4 + j2 + 2) % _NBUF
                out_wait(jn)
                gather_start(i + 2, jn)
                gather_wait(j)
                out_start(i, j)

        out_wait(2)
        gather_start(n_ch - 2, 2)
        gather_wait(0)
        out_start(n_ch - 4, 0)
        out_wait(3)
        gather_start(n_ch - 1, 3)
        gather_wait(1)
        out_start(n_ch - 3, 1)
        gather_wait(2)
        out_start(n_ch - 2, 2)
        gather_wait(3)
        out_start(n_ch - 1, 3)
        out_wait(0)
        out_wait(1)
        out_wait(2)
        out_wait(3)

    return k(q, idx_flat)


def kernel(inputs, table, W, b):
    Bsz, L = inputs.shape
    F = W.shape[1]
    q = _project_table(table, W, b.reshape(1, F))
    flat = inputs.reshape(Bsz * L).astype(jnp.int32)
    out = _sc_gather(q, flat)
    return out.reshape(Bsz, L, F)


# speedup vs baseline: 9.4346x; 1.0161x over previous
import functools

import jax
import jax.numpy as jnp
from jax import lax
from jax.experimental import pallas as pl
from jax.experimental.pallas import tpu as pltpu
from jax.experimental.pallas import tpu_sc as plsc


def _project_table(table, W, b2):
    V, D = table.shape
    F = W.shape[1]
    R = 10000

    def body(t_ref, w_ref, b_ref, o_ref):
        acc = jnp.dot(t_ref[...], w_ref[...], preferred_element_type=jnp.float32)
        o_ref[...] = jnp.tanh(acc + b_ref[...])

    return pl.pallas_call(
        body,
        grid=(V // R,),
        in_specs=[
            pl.BlockSpec((R, D), lambda i: (i, 0)),
            pl.BlockSpec((D, F), lambda i: (0, 0)),
            pl.BlockSpec((1, F), lambda i: (0, 0)),
        ],
        out_specs=pl.BlockSpec((R, F), lambda i: (i, 0)),
        out_shape=jax.ShapeDtypeStruct((V, F), jnp.float32),
    )(table, W, b2)


_NC = 2
_NS = 16
_NW = _NC * _NS
_CH = 200
_NBUF = 4


def _sc_gather(q, idx_flat):
    n = idx_flat.shape[0]
    F = q.shape[1]
    b_per_w = n // _NW
    n_ch = b_per_w // _CH
    mesh = plsc.VectorSubcoreMesh(core_axis_name="c", subcore_axis_name="s")

    @functools.partial(
        pl.kernel,
        mesh=mesh,
        out_type=jax.ShapeDtypeStruct((n, F), q.dtype),
        scratch_types=(
            [pltpu.VMEM((b_per_w,), jnp.int32)]
            + [pltpu.VMEM((_CH, F), jnp.float32) for _ in range(_NBUF)]
            + [pltpu.SemaphoreType.DMA for _ in range(2 * _NBUF)]
        ),
    )
    def k(q_hbm, i_hbm, o_hbm, idx_v, b0, b1, b2, b3, g0, g1, g2, g3, s0, s1, s2, s3):
        bufs = (b0, b1, b2, b3)
        gsem = (g0, g1, g2, g3)
        osem = (s0, s1, s2, s3)
        wid = lax.axis_index("s") * _NC + lax.axis_index("c")
        base = wid * b_per_w
        pltpu.sync_copy(i_hbm.at[pl.ds(base, b_per_w)], idx_v)

        def gather_start(c, j):
            pltpu.async_copy(q_hbm.at[idx_v.at[pl.ds(c * _CH, _CH)]], bufs[j], gsem[j])

        def gather_wait(j):
            pltpu.make_async_copy(
                q_hbm.at[idx_v.at[pl.ds(0, _CH)]], bufs[j], gsem[j]
            ).wait()

        def out_start(c, j):
            pltpu.async_copy(bufs[j], o_hbm.at[pl.ds(base + c * _CH, _CH)], osem[j])

        def out_wait(j):
            pltpu.make_async_copy(
                bufs[j], o_hbm.at[pl.ds(base, _CH)], osem[j]
            ).wait()

        gather_start(0, 0)
        gather_start(1, 1)
        gather_start(2, 2)
        gather_wait(0)
        out_start(0, 0)
        gather_start(3, 3)
        gather_wait(1)
        out_start(1, 1)
        out_wait(0)
        gather_start(4, 0)
        gather_wait(2)
        out_start(2, 2)
        out_wait(1)
        gather_start(5, 1)
        gather_wait(3)
        out_start(3, 3)

        @pl.loop(4, n_ch - 4, step=_NBUF)
        def _(c0):
            for j2 in range(_NBUF):
                i = c0 + j2
                j = (4 + j2) % _NBUF
                jn = (4 + j2 + 2) % _NBUF
                out_wait(jn)
                gather_start(i + 2, jn)
                gather_wait(j)
                out_start(i, j)

        out_wait(2)
        gather_start(n_ch - 2, 2)
        gather_wait(0)
        out_start(n_ch - 4, 0)
        out_wait(3)
        gather_start(n_ch - 1, 3)
        gather_wait(1)
        out_start(n_ch - 3, 1)
        gather_wait(2)
        out_start(n_ch - 2, 2)
        gather_wait(3)
        out_start(n_ch - 1, 3)
        out_wait(0)
        out_wait(1)
        out_wait(2)
        out_wait(3)

    return k(q, idx_flat)


def kernel(inputs, table, W, b):
    Bsz, L = inputs.shape
    F = W.shape[1]
    q = _project_table(table, W, b.reshape(1, F))
    flat = inputs.reshape(Bsz * L).astype(jnp.int32)
    out = _sc_gather(q, flat)
    return out.reshape(Bsz, L, F)
